# Initial kernel scaffold; baseline (speedup 1.0000x reference)
import jax
import jax.numpy as jnp
from jax.experimental import pallas as pl


def kernel(x, edge_index, edge_weight, conv_w, conv_b):
    raise NotImplementedError("write your pallas kernel here")


# speedup vs baseline: 3.7615x; 3.7615x over previous
import functools

import jax
import jax.numpy as jnp
from jax import lax
from jax.experimental import pallas as pl
from jax.experimental.pallas import tpu as pltpu
from jax.experimental.pallas import tpu_sc as plsc

_N = 10000
_E = 320000
_D = 128
_NC = 2
_NS = 16
_NW = _NC * _NS
_EPW = _E // _NW
_CH = 80
_NCHUNK = _EPW // _CH
_RPT = _N // _NS
_ZROWS = 125


def _sc_spmm(x, src, dst, w):
    mesh = plsc.VectorSubcoreMesh(core_axis_name="c", subcore_axis_name="s")

    @functools.partial(
        pl.kernel,
        out_type=jax.ShapeDtypeStruct((_NC, _N, _D), jnp.float32),
        mesh=mesh,
        scratch_types=[
            pltpu.VMEM((_CH,), jnp.int32),
            pltpu.VMEM((_CH,), jnp.int32),
            pltpu.VMEM((_CH,), jnp.float32),
            pltpu.VMEM((_CH, _D), jnp.float32),
            pltpu.VMEM((_ZROWS, _D), jnp.float32),
            pltpu.VMEM_SHARED((_N, _D), jnp.float32),
            pltpu.SemaphoreType.DMA,
        ],
        compiler_params=pltpu.CompilerParams(use_tc_tiling_on_sc=False,
                                             needs_layout_passes=False),
    )
    def k(x_hbm, src_hbm, dst_hbm, w_hbm, out_hbm, sidx, didx, wv, rows,
          zbuf, acc, sem):
        c = lax.axis_index("c")
        s = lax.axis_index("s")
        wid = s * _NC + c

        zero16 = jnp.zeros((16,), jnp.float32)

        def zrow(i, carry):
            for kk in range(_D // 16):
                zbuf[i, pl.ds(kk * 16, 16)] = zero16
            return carry

        lax.fori_loop(0, _ZROWS, zrow, 0)
        for j in range(_RPT // _ZROWS):
            pltpu.sync_copy(zbuf, acc.at[pl.ds(s * _RPT + j * _ZROWS, _ZROWS)])
        plsc.subcore_barrier()

        def chunk(cix, carry):
            base = wid * _EPW + cix * _CH
            pltpu.sync_copy(src_hbm.at[pl.ds(base, _CH)], sidx)
            pltpu.sync_copy(dst_hbm.at[pl.ds(base, _CH)], didx)
            pltpu.sync_copy(w_hbm.at[pl.ds(base, _CH)], wv)
            pltpu.async_copy(x_hbm.at[sidx], rows, sem).wait()

            def edge(e, ecarry):
                wb = plsc.load_gather(wv, [jnp.full((16,), e, jnp.int32)])
                for kk in range(_D // 16):
                    sl = pl.ds(kk * 16, 16)
                    rows[e, sl] = rows[e, sl] * wb
                return ecarry

            lax.fori_loop(0, _CH, edge, 0)
            pltpu.sync_copy(rows, acc.at[didx], add=True)
            return carry

        lax.fori_loop(0, _NCHUNK, chunk, 0)
        plsc.subcore_barrier()

        for j in range(_RPT // _ZROWS):
            r0 = s * _RPT + j * _ZROWS
            pltpu.sync_copy(acc.at[pl.ds(r0, _ZROWS)],
                            out_hbm.at[c, pl.ds(r0, _ZROWS)])

    return k(x, src, dst, w)


def _combine_body(scal_ref, p_ref, x_ref, o_ref):
    w0 = scal_ref[0]
    w1 = scal_ref[1]
    b = scal_ref[2]
    o_ref[...] = (p_ref[0] + p_ref[1]) * w0 + x_ref[...] * w1 + b


def _combine(partials, x, scal):
    blk = 1000
    grid = (_N // blk,)
    return pl.pallas_call(
        _combine_body,
        grid=grid,
        in_specs=[
            pl.BlockSpec(memory_space=pltpu.SMEM),
            pl.BlockSpec((_NC, blk, _D), lambda i: (0, i, 0)),
            pl.BlockSpec((blk, _D), lambda i: (i, 0)),
        ],
        out_specs=pl.BlockSpec((blk, _D), lambda i: (i, 0)),
        out_shape=jax.ShapeDtypeStruct((_N, _D), jnp.float32),
    )(scal, partials, x)


def kernel(x, edge_index, edge_weight, conv_w, conv_b):
    dst = edge_index[0]
    src = edge_index[1]
    partials = _sc_spmm(x, src, dst, edge_weight)
    scal = jnp.stack([conv_w[0, 0, 0, 0], conv_w[0, 0, 0, 1], conv_b[0]])
    return _combine(partials, x, scal)


# speedup vs baseline: 10.3245x; 2.7448x over previous
import functools

import jax
import jax.numpy as jnp
from jax import lax
from jax.experimental import pallas as pl
from jax.experimental.pallas import tpu as pltpu
from jax.experimental.pallas import tpu_sc as plsc

_N = 10000
_E = 320000
_D = 128
_NC = 2
_NS = 16
_NW = _NC * _NS
_EPW = _E // _NW
_CH = 80
_NCHUNK = _EPW // _CH
_RPT = _N // _NS


def _sc_spmm(x, src, dst, w):
    mesh = plsc.VectorSubcoreMesh(core_axis_name="c", subcore_axis_name="s")

    @functools.partial(
        pl.kernel,
        out_type=jax.ShapeDtypeStruct((_NC, _N, _D), jnp.float32),
        mesh=mesh,
        scratch_types=[
            pltpu.VMEM((_NCHUNK, _CH), jnp.int32),
            pltpu.VMEM((_NCHUNK, _CH), jnp.int32),
            pltpu.VMEM((_NCHUNK, _CH), jnp.float32),
            pltpu.VMEM((_CH, _D), jnp.float32),
            pltpu.VMEM((_CH, _D), jnp.float32),
            pltpu.VMEM_SHARED((_N, _D), jnp.float32),
            pltpu.SemaphoreType.DMA,
            pltpu.SemaphoreType.DMA,
        ],
        compiler_params=pltpu.CompilerParams(use_tc_tiling_on_sc=False,
                                             needs_layout_passes=False),
    )
    def k(x_hbm, src_hbm, dst_hbm, w_hbm, out_hbm, sidx, didx, wv,
          rows0, rows1, acc, sem0, sem1):
        c = lax.axis_index("c")
        s = lax.axis_index("s")
        wid = s * _NC + c

        pltpu.sync_copy(src_hbm.at[wid], sidx)
        pltpu.sync_copy(dst_hbm.at[wid], didx)
        pltpu.sync_copy(w_hbm.at[wid], wv)

        zero16 = jnp.zeros((16,), jnp.float32)

        def zrow(i, carry):
            for kk in range(_D // 16):
                rows0[i, pl.ds(kk * 16, 16)] = zero16
            return carry

        lax.fori_loop(0, _CH, zrow, 0)
        for j in range(_RPT // _CH):
            pltpu.sync_copy(rows0, acc.at[pl.ds(s * _RPT + j * _CH, _CH)])
        tail = _RPT - (_RPT // _CH) * _CH
        if tail:
            pltpu.sync_copy(
                rows0.at[pl.ds(0, tail)],
                acc.at[pl.ds(s * _RPT + (_RPT // _CH) * _CH, tail)])
        plsc.subcore_barrier()

        def start_gather(cix, rows, sem):
            pltpu.async_copy(x_hbm.at[sidx.at[cix]], rows, sem)

        def wait_gather(rows, sem):
            pltpu.make_async_copy(x_hbm.at[sidx.at[0]], rows, sem).wait()

        def do_chunk(cix, rows):
            @plsc.parallel_loop(0, _CH, 1, unroll=2)
            def scale(e):
                wb = plsc.load_gather(
                    wv, [jnp.full((16,), cix, jnp.int32),
                         jnp.full((16,), e, jnp.int32)])
                for kk in range(_D // 16):
                    sl = pl.ds(kk * 16, 16)
                    rows[e, sl] = rows[e, sl] * wb

            pltpu.sync_copy(rows, acc.at[didx.at[cix]], add=True)

        start_gather(0, rows0, sem0)

        def pair(i, carry):
            c0 = 2 * i
            start_gather(c0 + 1, rows1, sem1)
            wait_gather(rows0, sem0)
            do_chunk(c0, rows0)
            start_gather(c0 + 2, rows0, sem0)
            wait_gather(rows1, sem1)
            do_chunk(c0 + 1, rows1)
            return carry

        lax.fori_loop(0, (_NCHUNK - 1) // 2, pair, 0)
        wait_gather(rows0, sem0)
        do_chunk(_NCHUNK - 1, rows0)
        plsc.subcore_barrier()

        r0 = s * _RPT
        pltpu.sync_copy(acc.at[pl.ds(r0, _RPT)],
                        out_hbm.at[c, pl.ds(r0, _RPT)])

    return k(x, src.reshape(_NW, _NCHUNK, _CH), dst.reshape(_NW, _NCHUNK, _CH),
             w.reshape(_NW, _NCHUNK, _CH))


def _combine_body(scal_ref, p_ref, x_ref, o_ref):
    w0 = scal_ref[0]
    w1 = scal_ref[1]
    b = scal_ref[2]
    o_ref[...] = (p_ref[0] + p_ref[1]) * w0 + x_ref[...] * w1 + b


def _combine(partials, x, scal):
    blk = 1000
    grid = (_N // blk,)
    return pl.pallas_call(
        _combine_body,
        grid=grid,
        in_specs=[
            pl.BlockSpec(memory_space=pltpu.SMEM),
            pl.BlockSpec((_NC, blk, _D), lambda i: (0, i, 0)),
            pl.BlockSpec((blk, _D), lambda i: (i, 0)),
        ],
        out_specs=pl.BlockSpec((blk, _D), lambda i: (i, 0)),
        out_shape=jax.ShapeDtypeStruct((_N, _D), jnp.float32),
    )(scal, partials, x)


def kernel(x, edge_index, edge_weight, conv_w, conv_b):
    dst = edge_index[0]
    src = edge_index[1]
    partials = _sc_spmm(x, src, dst, edge_weight)
    scal = jnp.stack([conv_w[0, 0, 0, 0], conv_w[0, 0, 0, 1], conv_b[0]])
    return _combine(partials, x, scal)
